# Initial kernel scaffold; baseline (speedup 1.0000x reference)
import jax
import jax.numpy as jnp
from jax.experimental import pallas as pl


def kernel(constraint_x, variable_x, edge_attr, cons_shift, cons_scale, cons_W1, cons_b1, cons_W2, cons_b2, var_shift, var_scale, var_W1, var_b1, var_W2, var_b2, edge_shift, edge_scale, Wl_c2v_0, bl_c2v_0, Wr_c2v_0, Wl_v2c_0, bl_v2c_0, Wr_v2c_0, Wl_c2v_1, bl_c2v_1, Wr_c2v_1, Wl_v2c_1, bl_v2c_1, Wr_v2c_1, edge_index):
    raise NotImplementedError("write your pallas kernel here")


# speedup vs baseline: 4.2997x; 4.2997x over previous
import functools

import jax
import jax.numpy as jnp
from jax import lax
from jax.experimental import pallas as pl
from jax.experimental.pallas import tpu as pltpu
from jax.experimental.pallas import tpu_sc as plsc

N_NODES = 50000
N_EDGES = 800000
EMB = 64

CH = 256
CH_ROWS = 2
CH_LANES = 128
N_CHUNKS = N_EDGES // CH

HALF = N_NODES // 2
ROWS_PER_TILE = 1568
ACC_ROWS = 16 * ROWS_PER_TILE
DUMMY = 25080
CNT_W = 8

_MESH = plsc.VectorSubcoreMesh(core_axis_name="c", subcore_axis_name="s")
_SC_PARAMS = pltpu.CompilerParams(use_tc_tiling_on_sc=False)


def _local_idx(v, base):
    lv = v - base
    oob = (lv < 0) | (lv >= HALF)
    return jnp.where(oob, DUMMY, lv)


def _stage_local_idx(sidx_v, lidx_v, base):
    for r in range(CH_ROWS):
        for c in range(CH_LANES // 16):
            v = sidx_v[r, pl.ds(c * 16, 16)]
            lidx_v[r, pl.ds(c * 16, 16)] = _local_idx(v, base)


def _agg_kernel(x_hbm, gidx_hbm, sidx_hbm, zeros_hbm, out_hbm,
                gidx_v, sidx_v, lidx_v, rows_v, acc_sp, gsem):
    sc = lax.axis_index("c")
    t = lax.axis_index("s")
    base = sc * HALF

    pltpu.sync_copy(zeros_hbm, acc_sp.at[pl.ds(t * ROWS_PER_TILE, ROWS_PER_TILE)])
    plsc.subcore_barrier()

    def body(i, carry):
        k = t + 16 * i

        @pl.when(k < N_CHUNKS)
        def _():
            pltpu.sync_copy(gidx_hbm.at[k], gidx_v)
            pltpu.sync_copy(sidx_hbm.at[k], sidx_v)
            _stage_local_idx(sidx_v, lidx_v, base)
            copies = [
                pltpu.async_copy(x_hbm.at[gidx_v.at[r]],
                                 rows_v.at[pl.ds(r * CH_LANES, CH_LANES)], gsem)
                for r in range(CH_ROWS)
            ]
            for d in copies:
                d.wait()
            for r in range(CH_ROWS):
                pltpu.sync_copy(rows_v.at[pl.ds(r * CH_LANES, CH_LANES)],
                                acc_sp.at[lidx_v.at[r]], add=True)

        return carry

    n_iter = N_CHUNKS // 16 + 1
    lax.fori_loop(0, n_iter, body, 0)
    plsc.subcore_barrier()

    @pl.when(t < 15)
    def _():
        pltpu.sync_copy(acc_sp.at[pl.ds(t * ROWS_PER_TILE, ROWS_PER_TILE)],
                        out_hbm.at[sc].at[pl.ds(t * ROWS_PER_TILE, ROWS_PER_TILE)])

    @pl.when(t == 15)
    def _():
        last = HALF - 15 * ROWS_PER_TILE
        pltpu.sync_copy(acc_sp.at[pl.ds(15 * ROWS_PER_TILE, last)],
                        out_hbm.at[sc].at[pl.ds(15 * ROWS_PER_TILE, last)])


def _aggregate(x, gidx_rs, sidx_rs):
    zeros = jnp.zeros((ROWS_PER_TILE, EMB), jnp.float32)
    run = pl.kernel(
        _agg_kernel,
        out_type=jax.ShapeDtypeStruct((2, HALF, EMB), jnp.float32),
        mesh=_MESH,
        scratch_types=[
            pltpu.VMEM((CH_ROWS, CH_LANES), jnp.int32),
            pltpu.VMEM((CH_ROWS, CH_LANES), jnp.int32),
            pltpu.VMEM((CH_ROWS, CH_LANES), jnp.int32),
            pltpu.VMEM((CH, EMB), jnp.float32),
            pltpu.VMEM_SHARED((ACC_ROWS, EMB), jnp.float32),
            pltpu.SemaphoreType.DMA,
        ],
        compiler_params=_SC_PARAMS,
    )
    out = run(x, gidx_rs, sidx_rs, zeros)
    return out.reshape(N_NODES, EMB)


def _count_kernel(src_hbm, dst_hbm, ones_hbm, zeros_hbm, out_hbm,
                  sidx_v, lidx_v, ones_v, cnt_sp):
    sc = lax.axis_index("c")
    t = lax.axis_index("s")
    base = sc * HALF

    pltpu.sync_copy(ones_hbm, ones_v)
    for d in range(2):
        pltpu.sync_copy(zeros_hbm,
                        cnt_sp.at[d].at[pl.ds(t * ROWS_PER_TILE, ROWS_PER_TILE)])
    plsc.subcore_barrier()

    for d, idx_hbm in ((0, dst_hbm), (1, src_hbm)):
        def body(i, carry, idx_hbm=idx_hbm, d=d):
            k = t + 16 * i

            @pl.when(k < N_CHUNKS)
            def _():
                pltpu.sync_copy(idx_hbm.at[k], sidx_v)
                _stage_local_idx(sidx_v, lidx_v, base)
                for r in range(CH_ROWS):
                    pltpu.sync_copy(ones_v.at[pl.ds(r * CH_LANES, CH_LANES)],
                                    cnt_sp.at[d].at[lidx_v.at[r]], add=True)

            return carry

        lax.fori_loop(0, N_CHUNKS // 16 + 1, body, 0)

    plsc.subcore_barrier()
    for d in range(2):
        @pl.when(t < 15)
        def _(d=d):
            pltpu.sync_copy(cnt_sp.at[d].at[pl.ds(t * ROWS_PER_TILE, ROWS_PER_TILE)],
                            out_hbm.at[d].at[sc].at[pl.ds(t * ROWS_PER_TILE, ROWS_PER_TILE)])

        @pl.when(t == 15)
        def _(d=d):
            last = HALF - 15 * ROWS_PER_TILE
            pltpu.sync_copy(cnt_sp.at[d].at[pl.ds(15 * ROWS_PER_TILE, last)],
                            out_hbm.at[d].at[sc].at[pl.ds(15 * ROWS_PER_TILE, last)])


def _counts(src_rs, dst_rs):
    ones = jnp.ones((CH, CNT_W), jnp.float32)
    zeros = jnp.zeros((ROWS_PER_TILE, CNT_W), jnp.float32)
    run = pl.kernel(
        _count_kernel,
        out_type=jax.ShapeDtypeStruct((2, 2, HALF, CNT_W), jnp.float32),
        mesh=_MESH,
        scratch_types=[
            pltpu.VMEM((CH_ROWS, CH_LANES), jnp.int32),
            pltpu.VMEM((CH_ROWS, CH_LANES), jnp.int32),
            pltpu.VMEM((CH, CNT_W), jnp.float32),
            pltpu.VMEM_SHARED((2, ACC_ROWS, CNT_W), jnp.float32),
        ],
        compiler_params=_SC_PARAMS,
    )
    out = run(src_rs, dst_rs, ones, zeros)
    return out[0].reshape(N_NODES, CNT_W), out[1].reshape(N_NODES, CNT_W)


ROW_B = 1000


def _embed_body(x_ref, sh_ref, sc_ref, W1_ref, b1_ref, W2_ref, b2_ref, o_ref):
    xb = (x_ref[...] + sh_ref[...]) * sc_ref[...]
    h = jnp.dot(xb, W1_ref[...], preferred_element_type=jnp.float32) + b1_ref[...]
    h = jnp.maximum(h, 0.0)
    o = jnp.dot(h, W2_ref[...], preferred_element_type=jnp.float32) + b2_ref[...]
    o_ref[...] = jnp.maximum(o, 0.0)


def _embed(x, sh, sc, W1, b1, W2, b2):
    n, f = x.shape
    full = lambda shape: pl.BlockSpec(shape, lambda i: (0, 0))
    return pl.pallas_call(
        _embed_body,
        grid=(n // ROW_B,),
        in_specs=[
            pl.BlockSpec((ROW_B, f), lambda i: (i, 0)),
            full((1, f)), full((1, f)),
            full((f, EMB)), full((1, EMB)),
            full((EMB, EMB)), full((1, EMB)),
        ],
        out_specs=pl.BlockSpec((ROW_B, EMB), lambda i: (i, 0)),
        out_shape=jax.ShapeDtypeStruct((n, EMB), jnp.float32),
    )(x, sh.reshape(1, f), sc.reshape(1, f), W1, b1.reshape(1, EMB),
      W2, b2.reshape(1, EMB))


def _combine_body(s_ref, c_ref, x_ref, Wl_ref, bl_ref, Wr_ref, o_ref):
    cnt = c_ref[...][:, 0:1]
    mean = s_ref[...] / jnp.maximum(cnt, 1.0)
    o = (jnp.dot(mean, Wl_ref[...], preferred_element_type=jnp.float32)
         + jnp.dot(x_ref[...], Wr_ref[...], preferred_element_type=jnp.float32)
         + bl_ref[...])
    o_ref[...] = jnp.maximum(o, 0.0)


def _combine(sums, cnt, x, Wl, bl, Wr):
    n = sums.shape[0]
    full = lambda shape: pl.BlockSpec(shape, lambda i: (0, 0))
    return pl.pallas_call(
        _combine_body,
        grid=(n // ROW_B,),
        in_specs=[
            pl.BlockSpec((ROW_B, EMB), lambda i: (i, 0)),
            pl.BlockSpec((ROW_B, CNT_W), lambda i: (i, 0)),
            pl.BlockSpec((ROW_B, EMB), lambda i: (i, 0)),
            full((EMB, EMB)), full((1, EMB)), full((EMB, EMB)),
        ],
        out_specs=pl.BlockSpec((ROW_B, EMB), lambda i: (i, 0)),
        out_shape=jax.ShapeDtypeStruct((n, EMB), jnp.float32),
    )(sums, cnt, x, Wl, bl.reshape(1, EMB), Wr)


def kernel(constraint_x, variable_x, edge_attr,
           cons_shift, cons_scale, cons_W1, cons_b1, cons_W2, cons_b2,
           var_shift, var_scale, var_W1, var_b1, var_W2, var_b2,
           edge_shift, edge_scale,
           Wl_c2v_0, bl_c2v_0, Wr_c2v_0, Wl_v2c_0, bl_v2c_0, Wr_v2c_0,
           Wl_c2v_1, bl_c2v_1, Wr_c2v_1, Wl_v2c_1, bl_v2c_1, Wr_v2c_1,
           edge_index):
    src_rs = edge_index[0].reshape(N_CHUNKS, CH_ROWS, CH_LANES)
    dst_rs = edge_index[1].reshape(N_CHUNKS, CH_ROWS, CH_LANES)

    c0 = _embed(constraint_x, cons_shift, cons_scale, cons_W1, cons_b1, cons_W2, cons_b2)
    v0 = _embed(variable_x, var_shift, var_scale, var_W1, var_b1, var_W2, var_b2)
    cnt_v, cnt_c = _counts(src_rs, dst_rs)

    vs0 = _aggregate(c0, src_rs, dst_rs)
    cs0 = _aggregate(v0, dst_rs, src_rs)
    v1 = _combine(vs0, cnt_v, v0, Wl_c2v_0, bl_c2v_0, Wr_c2v_0)
    c1 = _combine(cs0, cnt_c, c0, Wl_v2c_0, bl_v2c_0, Wr_v2c_0)

    vs1 = _aggregate(c1, src_rs, dst_rs)
    v2 = _combine(vs1, cnt_v, v1, Wl_c2v_1, bl_c2v_1, Wr_c2v_1)
    return v2


# speedup vs baseline: 7.2618x; 1.6889x over previous
import functools

import jax
import jax.numpy as jnp
from jax import lax
from jax.experimental import pallas as pl
from jax.experimental.pallas import tpu as pltpu
from jax.experimental.pallas import tpu_sc as plsc

N_NODES = 50000
N_EDGES = 800000
EMB = 64

CH = 128
N_CHUNKS = N_EDGES // CH

HALF = N_NODES // 2
ROWS_PER_TILE = 1568
ACC_ROWS = 25008
LAST_TILE_ROWS = ACC_ROWS - 15 * ROWS_PER_TILE
DUMMY = 25000
CNT_W = 8

_MESH = plsc.VectorSubcoreMesh(core_axis_name="c", subcore_axis_name="s")
_SC_PARAMS = pltpu.CompilerParams(use_tc_tiling_on_sc=False)


def _compute_lidx(sidx_v, lidx_v, base):
    for c in range(CH // 16):
        v = sidx_v[pl.ds(c * 16, 16)]
        lv = v - base
        oob = (lv < 0) | (lv >= HALF)
        lidx_v[0, pl.ds(c * 16, 16)] = jnp.where(oob, DUMMY, lv)


def _agg_body(with_counts, x_hbm, gidx_hbm, sidx_hbm, zeros_hbm, zcnt_hbm,
              ones_hbm, out_hbm, cnt_out_hbm, gidx_v, sidx_v, lidx_v, rows_v,
              ones_v, acc_sp, cnt_sp, isem, gsem, ssem):
    sc = lax.axis_index("c")
    t = lax.axis_index("s")
    base = sc * HALF

    def zero_stripe(z_hbm, dst_sp):
        @pl.when(t < 15)
        def _():
            pltpu.sync_copy(z_hbm,
                            dst_sp.at[pl.ds(t * ROWS_PER_TILE, ROWS_PER_TILE)])

        @pl.when(t == 15)
        def _():
            pltpu.sync_copy(z_hbm.at[pl.ds(0, LAST_TILE_ROWS)],
                            dst_sp.at[pl.ds(15 * ROWS_PER_TILE, LAST_TILE_ROWS)])

    zero_stripe(zeros_hbm, acc_sp)
    if with_counts:
        pltpu.sync_copy(ones_hbm, ones_v)
        zero_stripe(zcnt_hbm, cnt_sp)
    plsc.subcore_barrier()

    def chunk_of(i):
        return t + 16 * i

    def stage(i, b):
        k = chunk_of(i)

        @pl.when(k < N_CHUNKS)
        def _():
            pltpu.async_copy(gidx_hbm.at[k], gidx_v[b], isem[b])
            pltpu.async_copy(sidx_hbm.at[k], sidx_v[b], isem[b])

    def wait_stage(i, b):
        k = chunk_of(i)

        @pl.when(k < N_CHUNKS)
        def _():
            pltpu.make_async_copy(gidx_hbm.at[k], gidx_v[b], isem[b]).wait()
            pltpu.make_async_copy(sidx_hbm.at[k], sidx_v[b], isem[b]).wait()

    def fire_gather(i, b):
        k = chunk_of(i)

        @pl.when(k < N_CHUNKS)
        def _():
            _compute_lidx(sidx_v[b], lidx_v[b], base)
            pltpu.async_copy(x_hbm.at[gidx_v[b]], rows_v[b], gsem[b])

    def fire_scatter(i, b):
        k = chunk_of(i)

        @pl.when(k < N_CHUNKS)
        def _():
            pltpu.make_async_copy(x_hbm.at[gidx_v[b]], rows_v[b], gsem[b]).wait()
            pltpu.async_copy(rows_v[b], acc_sp.at[lidx_v[b].at[0]], ssem[b],
                             add=True)
            if with_counts:
                pltpu.async_copy(ones_v, cnt_sp.at[lidx_v[b].at[0]], ssem[b],
                                 add=True)

    def wait_scatter(i, b):
        k = chunk_of(i)

        @pl.when(k < N_CHUNKS)
        def _():
            pltpu.make_async_copy(rows_v[b], acc_sp.at[lidx_v[b].at[0]],
                                  ssem[b]).wait()
            if with_counts:
                pltpu.make_async_copy(ones_v, cnt_sp.at[lidx_v[b].at[0]],
                                      ssem[b]).wait()

    n_slots = (N_CHUNKS + 15) // 16
    n_j = n_slots // 2 + 2

    stage(0, 0)

    def body(j, carry):
        for b in (0, 1):
            i = 2 * j + b

            @pl.when(i >= 2)
            def _(i=i, b=b):
                wait_scatter(i - 2, b)

            wait_stage(i, b)
            fire_gather(i, b)

            @pl.when(i >= 1)
            def _(i=i, b=b):
                fire_scatter(i - 1, 1 - b)

            stage(i + 1, 1 - b)
        return carry

    lax.fori_loop(0, n_j, body, 0)
    plsc.subcore_barrier()

    def writeback(src_sp, dst_hbm):
        @pl.when(t < 15)
        def _():
            pltpu.sync_copy(src_sp.at[pl.ds(t * ROWS_PER_TILE, ROWS_PER_TILE)],
                            dst_hbm.at[sc].at[pl.ds(t * ROWS_PER_TILE, ROWS_PER_TILE)])

        @pl.when(t == 15)
        def _():
            last = HALF - 15 * ROWS_PER_TILE
            pltpu.sync_copy(src_sp.at[pl.ds(15 * ROWS_PER_TILE, last)],
                            dst_hbm.at[sc].at[pl.ds(15 * ROWS_PER_TILE, last)])

    writeback(acc_sp, out_hbm)
    if with_counts:
        writeback(cnt_sp, cnt_out_hbm)


def _aggregate(x, gidx_rs, sidx_rs, with_counts):
    zeros = jnp.zeros((ROWS_PER_TILE, EMB), jnp.float32)
    zcnt = jnp.zeros((ROWS_PER_TILE, CNT_W), jnp.float32)
    ones = jnp.ones((CH, CNT_W), jnp.float32)
    out_type = jax.ShapeDtypeStruct((2, HALF, EMB), jnp.float32)
    dbuf = lambda shape, dt: [pltpu.VMEM(shape, dt), pltpu.VMEM(shape, dt)]
    sems = lambda: [pltpu.SemaphoreType.DMA, pltpu.SemaphoreType.DMA]
    scratch = (dbuf((CH,), jnp.int32) +
               dbuf((CH,), jnp.int32) +
               dbuf((1, CH), jnp.int32) +
               dbuf((CH, EMB), jnp.float32))
    if with_counts:
        out_type = (out_type, jax.ShapeDtypeStruct((2, HALF, CNT_W), jnp.float32))
        scratch += [pltpu.VMEM((CH, CNT_W), jnp.float32),
                    pltpu.VMEM_SHARED((ACC_ROWS, EMB), jnp.float32),
                    pltpu.VMEM_SHARED((ACC_ROWS, CNT_W), jnp.float32)]
    else:
        scratch += [pltpu.VMEM_SHARED((ACC_ROWS, EMB), jnp.float32)]
    scratch += sems() + sems() + sems()

    def kern(x_hbm, gidx_hbm, sidx_hbm, zeros_hbm, zcnt_hbm, ones_hbm, *refs):
        if with_counts:
            (out_hbm, cnt_out_hbm, g0, g1, s0, s1, l0, l1, r0, r1,
             ones_v, acc_sp, cnt_sp, i0, i1, gs0, gs1, ss0, ss1) = refs
        else:
            (out_hbm, g0, g1, s0, s1, l0, l1, r0, r1,
             acc_sp, i0, i1, gs0, gs1, ss0, ss1) = refs
            cnt_out_hbm = ones_v = cnt_sp = None
        _agg_body(with_counts, x_hbm, gidx_hbm, sidx_hbm, zeros_hbm, zcnt_hbm,
                  ones_hbm, out_hbm, cnt_out_hbm, (g0, g1), (s0, s1),
                  (l0, l1), (r0, r1), ones_v, acc_sp, cnt_sp,
                  (i0, i1), (gs0, gs1), (ss0, ss1))

    run = pl.kernel(
        kern,
        out_type=out_type,
        mesh=_MESH,
        scratch_types=scratch,
        compiler_params=_SC_PARAMS,
    )
    out = run(x, gidx_rs, sidx_rs, zeros, zcnt, ones)
    if with_counts:
        return out[0].reshape(N_NODES, EMB), out[1].reshape(N_NODES, CNT_W)
    return out.reshape(N_NODES, EMB)


ROW_B = 1000


def _embed_body(x_ref, sh_ref, sc_ref, W1_ref, b1_ref, W2_ref, b2_ref, o_ref):
    xb = (x_ref[...] + sh_ref[...]) * sc_ref[...]
    h = jnp.dot(xb, W1_ref[...], preferred_element_type=jnp.float32) + b1_ref[...]
    h = jnp.maximum(h, 0.0)
    o = jnp.dot(h, W2_ref[...], preferred_element_type=jnp.float32) + b2_ref[...]
    o_ref[...] = jnp.maximum(o, 0.0)


def _embed(x, sh, sc, W1, b1, W2, b2):
    n, f = x.shape
    full = lambda shape: pl.BlockSpec(shape, lambda i: (0, 0))
    return pl.pallas_call(
        _embed_body,
        grid=(n // ROW_B,),
        in_specs=[
            pl.BlockSpec((ROW_B, f), lambda i: (i, 0)),
            full((1, f)), full((1, f)),
            full((f, EMB)), full((1, EMB)),
            full((EMB, EMB)), full((1, EMB)),
        ],
        out_specs=pl.BlockSpec((ROW_B, EMB), lambda i: (i, 0)),
        out_shape=jax.ShapeDtypeStruct((n, EMB), jnp.float32),
    )(x, sh.reshape(1, f), sc.reshape(1, f), W1, b1.reshape(1, EMB),
      W2, b2.reshape(1, EMB))


def _combine_body(s_ref, c_ref, x_ref, Wl_ref, bl_ref, Wr_ref, o_ref):
    cnt = c_ref[...][:, 0:1]
    mean = s_ref[...] / jnp.maximum(cnt, 1.0)
    o = (jnp.dot(mean, Wl_ref[...], preferred_element_type=jnp.float32)
         + jnp.dot(x_ref[...], Wr_ref[...], preferred_element_type=jnp.float32)
         + bl_ref[...])
    o_ref[...] = jnp.maximum(o, 0.0)


def _combine(sums, cnt, x, Wl, bl, Wr):
    n = sums.shape[0]
    full = lambda shape: pl.BlockSpec(shape, lambda i: (0, 0))
    return pl.pallas_call(
        _combine_body,
        grid=(n // ROW_B,),
        in_specs=[
            pl.BlockSpec((ROW_B, EMB), lambda i: (i, 0)),
            pl.BlockSpec((ROW_B, CNT_W), lambda i: (i, 0)),
            pl.BlockSpec((ROW_B, EMB), lambda i: (i, 0)),
            full((EMB, EMB)), full((1, EMB)), full((EMB, EMB)),
        ],
        out_specs=pl.BlockSpec((ROW_B, EMB), lambda i: (i, 0)),
        out_shape=jax.ShapeDtypeStruct((n, EMB), jnp.float32),
    )(sums, cnt, x, Wl, bl.reshape(1, EMB), Wr)


def kernel(constraint_x, variable_x, edge_attr,
           cons_shift, cons_scale, cons_W1, cons_b1, cons_W2, cons_b2,
           var_shift, var_scale, var_W1, var_b1, var_W2, var_b2,
           edge_shift, edge_scale,
           Wl_c2v_0, bl_c2v_0, Wr_c2v_0, Wl_v2c_0, bl_v2c_0, Wr_v2c_0,
           Wl_c2v_1, bl_c2v_1, Wr_c2v_1, Wl_v2c_1, bl_v2c_1, Wr_v2c_1,
           edge_index):
    src_rs = edge_index[0].reshape(N_CHUNKS, CH)
    dst_rs = edge_index[1].reshape(N_CHUNKS, CH)

    c0 = _embed(constraint_x, cons_shift, cons_scale, cons_W1, cons_b1, cons_W2, cons_b2)
    v0 = _embed(variable_x, var_shift, var_scale, var_W1, var_b1, var_W2, var_b2)

    vs0, cnt_v = _aggregate(c0, src_rs, dst_rs, with_counts=True)
    cs0, cnt_c = _aggregate(v0, dst_rs, src_rs, with_counts=True)
    v1 = _combine(vs0, cnt_v, v0, Wl_c2v_0, bl_c2v_0, Wr_c2v_0)
    c1 = _combine(cs0, cnt_c, c0, Wl_v2c_0, bl_v2c_0, Wr_v2c_0)

    vs1 = _aggregate(c1, src_rs, dst_rs, with_counts=False)
    v2 = _combine(vs1, cnt_v, v1, Wl_c2v_1, bl_c2v_1, Wr_c2v_1)
    return v2


# speedup vs baseline: 10.7614x; 1.4819x over previous
import functools

import jax
import jax.numpy as jnp
from jax import lax
from jax.experimental import pallas as pl
from jax.experimental.pallas import tpu as pltpu
from jax.experimental.pallas import tpu_sc as plsc

N_NODES = 50000
N_EDGES = 800000
EMB = 64

CH = 128

PCH = 640
P_CHUNKS = N_EDGES // PCH
CAP = 25728

HALF = N_NODES // 2
ROWS_PER_TILE = 1568
ACC_ROWS = 25008
LAST_TILE_ROWS = ACC_ROWS - 15 * ROWS_PER_TILE
DUMMY = 25000
CNT_W = 8

_MESH = plsc.VectorSubcoreMesh(core_axis_name="c", subcore_axis_name="s")
_SC_PARAMS = pltpu.CompilerParams(use_tc_tiling_on_sc=False)
_SC_PARAMS_NL = pltpu.CompilerParams(use_tc_tiling_on_sc=False,
                                     needs_layout_passes=False)


def _partition_body(src_hbm, dst_hbm, glist_hbm, llist_hbm, len_hbm,
                    gbuf, sbuf, g0, l0, g1, l1, lenv):
    sc = lax.axis_index("c")
    t = lax.axis_index("s")
    wid = sc * 16 + t

    for d in (0, 1):
        ga, sa = (src_hbm, dst_hbm) if d == 0 else (dst_hbm, src_hbm)

        def chunk_body(i, ns, ga=ga, sa=sa):
            n0, n1 = ns
            k = wid + 32 * i
            pltpu.sync_copy(ga.at[k], gbuf)
            pltpu.sync_copy(sa.at[k], sbuf)
            lanes = lax.iota(jnp.int32, 16)
            for c in range(PCH // 16):
                g = gbuf[pl.ds(c * 16, 16)]
                s = sbuf[pl.ds(c * 16, 16)]
                m0 = s < HALF
                inc0 = plsc.cumsum(jnp.where(m0, 1, 0).astype(jnp.int32))
                k0 = jnp.sum(jnp.where(m0, 1, 0).astype(jnp.int32))
                pos0 = jnp.where(m0, n0 + inc0 - 1, CAP + lanes)
                pos1 = jnp.where(m0, CAP + lanes, n1 + (lanes - inc0))
                plsc.store_scatter(g0, [pos0], g)
                plsc.store_scatter(l0, [pos0], s)
                plsc.store_scatter(g1, [pos1], g)
                plsc.store_scatter(l1, [pos1], s - HALF)
                n0 = n0 + k0
                n1 = n1 + (16 - k0)
            return (n0, n1)

        n_my = (P_CHUNKS - wid + 31) // 32
        n0, n1 = lax.fori_loop(0, n_my, chunk_body,
                               (jnp.int32(0), jnp.int32(0)))

        dummy_l = jnp.full((16,), DUMMY, jnp.int32)
        dummy_g = jnp.zeros((16,), jnp.int32)
        lanes = lax.iota(jnp.int32, 16)
        for j in range(CH // 16):
            plsc.store_scatter(g0, [n0 + 16 * j + lanes], dummy_g)
            plsc.store_scatter(l0, [n0 + 16 * j + lanes], dummy_l)
            plsc.store_scatter(g1, [n1 + 16 * j + lanes], dummy_g)
            plsc.store_scatter(l1, [n1 + 16 * j + lanes], dummy_l)
        nch0 = (n0 + CH - 1) // CH
        nch1 = (n1 + CH - 1) // CH

        pltpu.sync_copy(g0.at[pl.ds(0, CAP)], glist_hbm.at[d].at[wid].at[0])
        pltpu.sync_copy(l0.at[pl.ds(0, CAP)], llist_hbm.at[d].at[wid].at[0])
        pltpu.sync_copy(g1.at[pl.ds(0, CAP)], glist_hbm.at[d].at[wid].at[1])
        pltpu.sync_copy(l1.at[pl.ds(0, CAP)], llist_hbm.at[d].at[wid].at[1])
        lenv[pl.ds(0, 16)] = jnp.zeros((16,), jnp.int32) + nch0
        pltpu.sync_copy(lenv, len_hbm.at[d].at[wid].at[0])
        lenv[pl.ds(0, 16)] = jnp.zeros((16,), jnp.int32) + nch1
        pltpu.sync_copy(lenv, len_hbm.at[d].at[wid].at[1])


def _partition(src_rs, dst_rs):
    run = pl.kernel(
        _partition_body,
        out_type=(jax.ShapeDtypeStruct((2, 32, 2, CAP), jnp.int32),
                  jax.ShapeDtypeStruct((2, 32, 2, CAP), jnp.int32),
                  jax.ShapeDtypeStruct((2, 32, 2, 16), jnp.int32)),
        mesh=_MESH,
        scratch_types=[
            pltpu.VMEM((PCH,), jnp.int32),
            pltpu.VMEM((PCH,), jnp.int32),
            pltpu.VMEM((CAP + 16,), jnp.int32),
            pltpu.VMEM((CAP + 16,), jnp.int32),
            pltpu.VMEM((CAP + 16,), jnp.int32),
            pltpu.VMEM((CAP + 16,), jnp.int32),
            pltpu.VMEM((16,), jnp.int32),
        ],
        compiler_params=_SC_PARAMS_NL,
    )
    return run(src_rs, dst_rs)


def _agg_body(with_counts, x_hbm, glist_hbm, llist_hbm, len_hbm, zeros_hbm,
              zcnt_hbm, ones_hbm, out_hbm, cnt_out_hbm, gidx_v, lidx_v,
              rows_v, ones_v, lenv, acc_sp, cnt_sp, isem, gsem, ssem):
    sc = lax.axis_index("c")
    t = lax.axis_index("s")

    def zero_stripe(z_hbm, dst_sp):
        @pl.when(t < 15)
        def _():
            pltpu.sync_copy(z_hbm,
                            dst_sp.at[pl.ds(t * ROWS_PER_TILE, ROWS_PER_TILE)])

        @pl.when(t == 15)
        def _():
            pltpu.sync_copy(z_hbm.at[pl.ds(0, LAST_TILE_ROWS)],
                            dst_sp.at[pl.ds(15 * ROWS_PER_TILE, LAST_TILE_ROWS)])

    zero_stripe(zeros_hbm, acc_sp)
    if with_counts:
        pltpu.sync_copy(ones_hbm, ones_v)
        zero_stripe(zcnt_hbm, cnt_sp)

    pltpu.sync_copy(len_hbm.at[2 * t].at[sc], lenv)
    nch0 = lenv[pl.ds(0, 16)][0]
    pltpu.sync_copy(len_hbm.at[2 * t + 1].at[sc], lenv)
    nch1 = lenv[pl.ds(0, 16)][0]
    ntot = nch0 + nch1
    plsc.subcore_barrier()

    def loc(i):
        p = jnp.where(i < nch0, 2 * t, 2 * t + 1)
        j = jnp.where(i < nch0, i, i - nch0)
        return p, j

    def stage(i, b, lb):
        @pl.when(i < ntot)
        def _():
            p, j = loc(i)
            pltpu.async_copy(glist_hbm.at[p].at[sc].at[pl.ds(j * CH, CH)],
                             gidx_v[b], isem[b])
            pltpu.async_copy(llist_hbm.at[p].at[sc].at[pl.ds(j * CH, CH)],
                             lidx_v[lb], isem[b])

    def wait_stage(i, b, lb):
        @pl.when(i < ntot)
        def _():
            p, j = loc(i)
            pltpu.make_async_copy(glist_hbm.at[p].at[sc].at[pl.ds(j * CH, CH)],
                                  gidx_v[b], isem[b]).wait()
            pltpu.make_async_copy(llist_hbm.at[p].at[sc].at[pl.ds(j * CH, CH)],
                                  lidx_v[lb], isem[b]).wait()

    def fire_gather(i, b):
        @pl.when(i < ntot)
        def _():
            pltpu.async_copy(x_hbm.at[gidx_v[b]], rows_v[b], gsem[b])

    def fire_scatter(i, b, lb):
        @pl.when(i < ntot)
        def _():
            pltpu.make_async_copy(x_hbm.at[gidx_v[b]], rows_v[b], gsem[b]).wait()
            pltpu.async_copy(rows_v[b], acc_sp.at[lidx_v[lb]], ssem[b],
                             add=True)
            if with_counts:
                pltpu.async_copy(ones_v, cnt_sp.at[lidx_v[lb]], ssem[b],
                                 add=True)

    def wait_scatter(i, b, lb):
        @pl.when(i < ntot)
        def _():
            pltpu.make_async_copy(rows_v[b], acc_sp.at[lidx_v[lb]],
                                  ssem[b]).wait()
            if with_counts:
                pltpu.make_async_copy(ones_v, cnt_sp.at[lidx_v[lb]],
                                      ssem[b]).wait()

    stage(0, 0, 0)

    def body(j, carry):
        for q in range(4):
            i = 4 * j + q
            b = q % 2

            @pl.when(i >= 2)
            def _(i=i, b=b, q=q):
                wait_scatter(i - 2, b, (q + 2) % 4)

            wait_stage(i, b, q)
            fire_gather(i, b)

            @pl.when(i >= 1)
            def _(i=i, b=b, q=q):
                fire_scatter(i - 1, 1 - b, (q + 3) % 4)

            stage(i + 1, 1 - b, (q + 1) % 4)
        return carry

    lax.fori_loop(0, ntot // 4 + 2, body, 0)
    plsc.subcore_barrier()

    def writeback(src_sp, dst_hbm):
        @pl.when(t < 15)
        def _():
            pltpu.sync_copy(src_sp.at[pl.ds(t * ROWS_PER_TILE, ROWS_PER_TILE)],
                            dst_hbm.at[sc].at[pl.ds(t * ROWS_PER_TILE, ROWS_PER_TILE)])

        @pl.when(t == 15)
        def _():
            last = HALF - 15 * ROWS_PER_TILE
            pltpu.sync_copy(src_sp.at[pl.ds(15 * ROWS_PER_TILE, last)],
                            dst_hbm.at[sc].at[pl.ds(15 * ROWS_PER_TILE, last)])

    writeback(acc_sp, out_hbm)
    if with_counts:
        writeback(cnt_sp, cnt_out_hbm)


def _aggregate(x, glist_d, llist_d, len_d, with_counts):
    zeros = jnp.zeros((ROWS_PER_TILE, EMB), jnp.float32)
    zcnt = jnp.zeros((ROWS_PER_TILE, CNT_W), jnp.float32)
    ones = jnp.ones((CH, CNT_W), jnp.float32)
    out_type = jax.ShapeDtypeStruct((2, HALF, EMB), jnp.float32)
    dbuf = lambda shape, dt: [pltpu.VMEM(shape, dt), pltpu.VMEM(shape, dt)]
    sems = lambda: [pltpu.SemaphoreType.DMA, pltpu.SemaphoreType.DMA]
    scratch = (dbuf((CH,), jnp.int32) +
               dbuf((CH,), jnp.int32) * 2 +
               dbuf((CH, EMB), jnp.float32))
    if with_counts:
        out_type = (out_type, jax.ShapeDtypeStruct((2, HALF, CNT_W), jnp.float32))
        scratch += [pltpu.VMEM((CH, CNT_W), jnp.float32),
                    pltpu.VMEM((16,), jnp.int32),
                    pltpu.VMEM_SHARED((ACC_ROWS, EMB), jnp.float32),
                    pltpu.VMEM_SHARED((ACC_ROWS, CNT_W), jnp.float32)]
    else:
        scratch += [pltpu.VMEM((16,), jnp.int32),
                    pltpu.VMEM_SHARED((ACC_ROWS, EMB), jnp.float32)]
    scratch += sems() + sems() + sems()

    def kern(x_hbm, glist_hbm, llist_hbm, len_hbm, zeros_hbm, zcnt_hbm,
             ones_hbm, *refs):
        if with_counts:
            (out_hbm, cnt_out_hbm, g0, g1, l0, l1, l2, l3, r0, r1,
             ones_v, lenv, acc_sp, cnt_sp, i0, i1, gs0, gs1, ss0, ss1) = refs
        else:
            (out_hbm, g0, g1, l0, l1, l2, l3, r0, r1,
             lenv, acc_sp, i0, i1, gs0, gs1, ss0, ss1) = refs
            cnt_out_hbm = ones_v = cnt_sp = None
        _agg_body(with_counts, x_hbm, glist_hbm, llist_hbm, len_hbm,
                  zeros_hbm, zcnt_hbm, ones_hbm, out_hbm, cnt_out_hbm,
                  (g0, g1), (l0, l1, l2, l3), (r0, r1), ones_v, lenv, acc_sp, cnt_sp,
                  (i0, i1), (gs0, gs1), (ss0, ss1))

    run = pl.kernel(
        kern,
        out_type=out_type,
        mesh=_MESH,
        scratch_types=scratch,
        compiler_params=_SC_PARAMS,
    )
    out = run(x, glist_d, llist_d, len_d, zeros, zcnt, ones)
    if with_counts:
        return out[0].reshape(N_NODES, EMB), out[1].reshape(N_NODES, CNT_W)
    return out.reshape(N_NODES, EMB)


ROW_B = 1000


def _embed_body(x_ref, sh_ref, sc_ref, W1_ref, b1_ref, W2_ref, b2_ref, o_ref):
    xb = (x_ref[...] + sh_ref[...]) * sc_ref[...]
    h = jnp.dot(xb, W1_ref[...], preferred_element_type=jnp.float32) + b1_ref[...]
    h = jnp.maximum(h, 0.0)
    o = jnp.dot(h, W2_ref[...], preferred_element_type=jnp.float32) + b2_ref[...]
    o_ref[...] = jnp.maximum(o, 0.0)


def _embed(x, sh, sc, W1, b1, W2, b2):
    n, f = x.shape
    full = lambda shape: pl.BlockSpec(shape, lambda i: (0, 0))
    return pl.pallas_call(
        _embed_body,
        grid=(n // ROW_B,),
        in_specs=[
            pl.BlockSpec((ROW_B, f), lambda i: (i, 0)),
            full((1, f)), full((1, f)),
            full((f, EMB)), full((1, EMB)),
            full((EMB, EMB)), full((1, EMB)),
        ],
        out_specs=pl.BlockSpec((ROW_B, EMB), lambda i: (i, 0)),
        out_shape=jax.ShapeDtypeStruct((n, EMB), jnp.float32),
    )(x, sh.reshape(1, f), sc.reshape(1, f), W1, b1.reshape(1, EMB),
      W2, b2.reshape(1, EMB))


def _combine_body(s_ref, c_ref, x_ref, Wl_ref, bl_ref, Wr_ref, o_ref):
    cnt = c_ref[...][:, 0:1]
    mean = s_ref[...] / jnp.maximum(cnt, 1.0)
    o = (jnp.dot(mean, Wl_ref[...], preferred_element_type=jnp.float32)
         + jnp.dot(x_ref[...], Wr_ref[...], preferred_element_type=jnp.float32)
         + bl_ref[...])
    o_ref[...] = jnp.maximum(o, 0.0)


def _combine(sums, cnt, x, Wl, bl, Wr):
    n = sums.shape[0]
    full = lambda shape: pl.BlockSpec(shape, lambda i: (0, 0))
    return pl.pallas_call(
        _combine_body,
        grid=(n // ROW_B,),
        in_specs=[
            pl.BlockSpec((ROW_B, EMB), lambda i: (i, 0)),
            pl.BlockSpec((ROW_B, CNT_W), lambda i: (i, 0)),
            pl.BlockSpec((ROW_B, EMB), lambda i: (i, 0)),
            full((EMB, EMB)), full((1, EMB)), full((EMB, EMB)),
        ],
        out_specs=pl.BlockSpec((ROW_B, EMB), lambda i: (i, 0)),
        out_shape=jax.ShapeDtypeStruct((n, EMB), jnp.float32),
    )(sums, cnt, x, Wl, bl.reshape(1, EMB), Wr)


def kernel(constraint_x, variable_x, edge_attr,
           cons_shift, cons_scale, cons_W1, cons_b1, cons_W2, cons_b2,
           var_shift, var_scale, var_W1, var_b1, var_W2, var_b2,
           edge_shift, edge_scale,
           Wl_c2v_0, bl_c2v_0, Wr_c2v_0, Wl_v2c_0, bl_v2c_0, Wr_v2c_0,
           Wl_c2v_1, bl_c2v_1, Wr_c2v_1, Wl_v2c_1, bl_v2c_1, Wr_v2c_1,
           edge_index):
    src_rs = edge_index[0].reshape(P_CHUNKS, PCH)
    dst_rs = edge_index[1].reshape(P_CHUNKS, PCH)

    glist, llist, lens = _partition(src_rs, dst_rs)
    c0 = _embed(constraint_x, cons_shift, cons_scale, cons_W1, cons_b1, cons_W2, cons_b2)
    v0 = _embed(variable_x, var_shift, var_scale, var_W1, var_b1, var_W2, var_b2)

    vs0, cnt_v = _aggregate(c0, glist[0], llist[0], lens[0], with_counts=True)
    cs0, cnt_c = _aggregate(v0, glist[1], llist[1], lens[1], with_counts=True)
    v1 = _combine(vs0, cnt_v, v0, Wl_c2v_0, bl_c2v_0, Wr_c2v_0)
    c1 = _combine(cs0, cnt_c, c0, Wl_v2c_0, bl_v2c_0, Wr_v2c_0)

    vs1 = _aggregate(c1, glist[0], llist[0], lens[0], with_counts=False)
    v2 = _combine(vs1, cnt_v, v1, Wl_c2v_1, bl_c2v_1, Wr_c2v_1)
    return v2


# speedup vs baseline: 11.9232x; 1.1080x over previous
import functools

import jax
import jax.numpy as jnp
from jax import lax
from jax.experimental import pallas as pl
from jax.experimental.pallas import tpu as pltpu
from jax.experimental.pallas import tpu_sc as plsc

N_NODES = 50000
N_EDGES = 800000
EMB = 64

CH = 128

PCH = 640
P_CHUNKS = N_EDGES // PCH
CAP = 25728

HALF = N_NODES // 2
ROWS_PER_TILE = 1568
ACC_ROWS = 25008
LAST_TILE_ROWS = ACC_ROWS - 15 * ROWS_PER_TILE
DUMMY = 25000
CNT_W = 8

_MESH = plsc.VectorSubcoreMesh(core_axis_name="c", subcore_axis_name="s")
_SC_PARAMS = pltpu.CompilerParams(use_tc_tiling_on_sc=False)
_SC_PARAMS_NL = pltpu.CompilerParams(use_tc_tiling_on_sc=False,
                                     needs_layout_passes=False)


def _partition_body(src_hbm, dst_hbm, glist0_hbm, llist0_hbm, len0_hbm,
                    glist1_hbm, llist1_hbm, len1_hbm,
                    gbuf, sbuf, g0, l0, g1, l1, lenv):
    sc = lax.axis_index("c")
    t = lax.axis_index("s")
    wid = sc * 16 + t

    for d in (0, 1):
        ga, sa = (src_hbm, dst_hbm) if d == 0 else (dst_hbm, src_hbm)
        glist_hbm = glist0_hbm if d == 0 else glist1_hbm
        llist_hbm = llist0_hbm if d == 0 else llist1_hbm
        len_hbm = len0_hbm if d == 0 else len1_hbm

        def chunk_body(i, ns, ga=ga, sa=sa):
            n0, n1 = ns
            k = wid + 32 * i
            pltpu.sync_copy(ga.at[k], gbuf)
            pltpu.sync_copy(sa.at[k], sbuf)
            lanes = lax.iota(jnp.int32, 16)
            for c in range(PCH // 16):
                g = gbuf[pl.ds(c * 16, 16)]
                s = sbuf[pl.ds(c * 16, 16)]
                m0 = s < HALF
                inc0 = plsc.cumsum(jnp.where(m0, 1, 0).astype(jnp.int32))
                k0 = jnp.sum(jnp.where(m0, 1, 0).astype(jnp.int32))
                pos0 = jnp.where(m0, n0 + inc0 - 1, CAP + lanes)
                pos1 = jnp.where(m0, CAP + lanes, n1 + (lanes - inc0))
                plsc.store_scatter(g0, [pos0], g)
                plsc.store_scatter(l0, [pos0], s)
                plsc.store_scatter(g1, [pos1], g)
                plsc.store_scatter(l1, [pos1], s - HALF)
                n0 = n0 + k0
                n1 = n1 + (16 - k0)
            return (n0, n1)

        n_my = (P_CHUNKS - wid + 31) // 32
        n0, n1 = lax.fori_loop(0, n_my, chunk_body,
                               (jnp.int32(0), jnp.int32(0)))

        dummy_l = jnp.full((16,), DUMMY, jnp.int32)
        dummy_g = jnp.zeros((16,), jnp.int32)
        lanes = lax.iota(jnp.int32, 16)
        for j in range(CH // 16):
            plsc.store_scatter(g0, [n0 + 16 * j + lanes], dummy_g)
            plsc.store_scatter(l0, [n0 + 16 * j + lanes], dummy_l)
            plsc.store_scatter(g1, [n1 + 16 * j + lanes], dummy_g)
            plsc.store_scatter(l1, [n1 + 16 * j + lanes], dummy_l)
        nch0 = (n0 + CH - 1) // CH
        nch1 = (n1 + CH - 1) // CH

        pltpu.sync_copy(g0.at[pl.ds(0, CAP)], glist_hbm.at[wid].at[0])
        pltpu.sync_copy(l0.at[pl.ds(0, CAP)], llist_hbm.at[wid].at[0])
        pltpu.sync_copy(g1.at[pl.ds(0, CAP)], glist_hbm.at[wid].at[1])
        pltpu.sync_copy(l1.at[pl.ds(0, CAP)], llist_hbm.at[wid].at[1])
        lenv[pl.ds(0, 16)] = jnp.zeros((16,), jnp.int32) + nch0
        pltpu.sync_copy(lenv, len_hbm.at[wid].at[0])
        lenv[pl.ds(0, 16)] = jnp.zeros((16,), jnp.int32) + nch1
        pltpu.sync_copy(lenv, len_hbm.at[wid].at[1])


def _partition(src_rs, dst_rs):
    run = pl.kernel(
        _partition_body,
        out_type=(jax.ShapeDtypeStruct((32, 2, CAP), jnp.int32),
                  jax.ShapeDtypeStruct((32, 2, CAP), jnp.int32),
                  jax.ShapeDtypeStruct((32, 2, 16), jnp.int32)) * 2,
        mesh=_MESH,
        scratch_types=[
            pltpu.VMEM((PCH,), jnp.int32),
            pltpu.VMEM((PCH,), jnp.int32),
            pltpu.VMEM((CAP + 16,), jnp.int32),
            pltpu.VMEM((CAP + 16,), jnp.int32),
            pltpu.VMEM((CAP + 16,), jnp.int32),
            pltpu.VMEM((CAP + 16,), jnp.int32),
            pltpu.VMEM((16,), jnp.int32),
        ],
        compiler_params=_SC_PARAMS_NL,
    )
    return run(src_rs, dst_rs)


def _agg_body(with_counts, x_hbm, glist_hbm, llist_hbm, len_hbm, zeros_hbm,
              zcnt_hbm, ones_hbm, out_hbm, cnt_out_hbm, gidx_v, lidx_v,
              rows_v, ones_v, lenv, acc_sp, cnt_sp, isem, gsem, ssem):
    sc = lax.axis_index("c")
    t = lax.axis_index("s")

    def zero_stripe(z_hbm, dst_sp):
        @pl.when(t < 15)
        def _():
            pltpu.sync_copy(z_hbm,
                            dst_sp.at[pl.ds(t * ROWS_PER_TILE, ROWS_PER_TILE)])

        @pl.when(t == 15)
        def _():
            pltpu.sync_copy(z_hbm.at[pl.ds(0, LAST_TILE_ROWS)],
                            dst_sp.at[pl.ds(15 * ROWS_PER_TILE, LAST_TILE_ROWS)])

    zero_stripe(zeros_hbm, acc_sp)
    if with_counts:
        pltpu.sync_copy(ones_hbm, ones_v)
        zero_stripe(zcnt_hbm, cnt_sp)

    pltpu.sync_copy(len_hbm.at[2 * t].at[sc], lenv)
    nch0 = lenv[pl.ds(0, 16)][0]
    pltpu.sync_copy(len_hbm.at[2 * t + 1].at[sc], lenv)
    nch1 = lenv[pl.ds(0, 16)][0]
    ntot = nch0 + nch1
    plsc.subcore_barrier()

    def loc(i):
        p = jnp.where(i < nch0, 2 * t, 2 * t + 1)
        j = jnp.where(i < nch0, i, i - nch0)
        return p, j

    def stage(i, b, lb):
        @pl.when(i < ntot)
        def _():
            p, j = loc(i)
            pltpu.async_copy(glist_hbm.at[p].at[sc].at[pl.ds(j * CH, CH)],
                             gidx_v[b], isem[b])
            pltpu.async_copy(llist_hbm.at[p].at[sc].at[pl.ds(j * CH, CH)],
                             lidx_v[lb], isem[b])

    def wait_stage(i, b, lb):
        @pl.when(i < ntot)
        def _():
            p, j = loc(i)
            pltpu.make_async_copy(glist_hbm.at[p].at[sc].at[pl.ds(j * CH, CH)],
                                  gidx_v[b], isem[b]).wait()
            pltpu.make_async_copy(llist_hbm.at[p].at[sc].at[pl.ds(j * CH, CH)],
                                  lidx_v[lb], isem[b]).wait()

    def fire_gather(i, b):
        @pl.when(i < ntot)
        def _():
            pltpu.async_copy(x_hbm.at[gidx_v[b]], rows_v[b], gsem[b])

    def fire_scatter(i, b, lb):
        @pl.when(i < ntot)
        def _():
            pltpu.make_async_copy(x_hbm.at[gidx_v[b]], rows_v[b], gsem[b]).wait()
            pltpu.async_copy(rows_v[b], acc_sp.at[lidx_v[lb]], ssem[b],
                             add=True)
            if with_counts:
                pltpu.async_copy(ones_v, cnt_sp.at[lidx_v[lb]], ssem[b],
                                 add=True)

    def wait_scatter(i, b, lb):
        @pl.when(i < ntot)
        def _():
            pltpu.make_async_copy(rows_v[b], acc_sp.at[lidx_v[lb]],
                                  ssem[b]).wait()
            if with_counts:
                pltpu.make_async_copy(ones_v, cnt_sp.at[lidx_v[lb]],
                                      ssem[b]).wait()

    stage(0, 0, 0)

    def body(j, carry):
        for q in range(4):
            i = 4 * j + q
            b = q % 2

            @pl.when(i >= 2)
            def _(i=i, b=b, q=q):
                wait_scatter(i - 2, b, (q + 2) % 4)

            wait_stage(i, b, q)
            fire_gather(i, b)

            @pl.when(i >= 1)
            def _(i=i, b=b, q=q):
                fire_scatter(i - 1, 1 - b, (q + 3) % 4)

            stage(i + 1, 1 - b, (q + 1) % 4)
        return carry

    lax.fori_loop(0, ntot // 4 + 2, body, 0)
    plsc.subcore_barrier()

    def writeback(src_sp, dst_hbm):
        @pl.when(t < 15)
        def _():
            pltpu.sync_copy(src_sp.at[pl.ds(t * ROWS_PER_TILE, ROWS_PER_TILE)],
                            dst_hbm.at[sc].at[pl.ds(t * ROWS_PER_TILE, ROWS_PER_TILE)])

        @pl.when(t == 15)
        def _():
            last = HALF - 15 * ROWS_PER_TILE
            pltpu.sync_copy(src_sp.at[pl.ds(15 * ROWS_PER_TILE, last)],
                            dst_hbm.at[sc].at[pl.ds(15 * ROWS_PER_TILE, last)])

    writeback(acc_sp, out_hbm)
    if with_counts:
        writeback(cnt_sp, cnt_out_hbm)


def _aggregate(x, glist_d, llist_d, len_d, with_counts):
    zeros = jnp.zeros((ROWS_PER_TILE, EMB), jnp.float32)
    zcnt = jnp.zeros((ROWS_PER_TILE, CNT_W), jnp.float32)
    ones = jnp.ones((CH, CNT_W), jnp.float32)
    out_type = jax.ShapeDtypeStruct((2, HALF, EMB), jnp.float32)
    dbuf = lambda shape, dt: [pltpu.VMEM(shape, dt), pltpu.VMEM(shape, dt)]
    sems = lambda: [pltpu.SemaphoreType.DMA, pltpu.SemaphoreType.DMA]
    scratch = (dbuf((CH,), jnp.int32) +
               dbuf((CH,), jnp.int32) * 2 +
               dbuf((CH, EMB), jnp.float32))
    if with_counts:
        out_type = (out_type, jax.ShapeDtypeStruct((2, HALF, CNT_W), jnp.float32))
        scratch += [pltpu.VMEM((CH, CNT_W), jnp.float32),
                    pltpu.VMEM((16,), jnp.int32),
                    pltpu.VMEM_SHARED((ACC_ROWS, EMB), jnp.float32),
                    pltpu.VMEM_SHARED((ACC_ROWS, CNT_W), jnp.float32)]
    else:
        scratch += [pltpu.VMEM((16,), jnp.int32),
                    pltpu.VMEM_SHARED((ACC_ROWS, EMB), jnp.float32)]
    scratch += sems() + sems() + sems()

    def kern(x_hbm, glist_hbm, llist_hbm, len_hbm, zeros_hbm, zcnt_hbm,
             ones_hbm, *refs):
        if with_counts:
            (out_hbm, cnt_out_hbm, g0, g1, l0, l1, l2, l3, r0, r1,
             ones_v, lenv, acc_sp, cnt_sp, i0, i1, gs0, gs1, ss0, ss1) = refs
        else:
            (out_hbm, g0, g1, l0, l1, l2, l3, r0, r1,
             lenv, acc_sp, i0, i1, gs0, gs1, ss0, ss1) = refs
            cnt_out_hbm = ones_v = cnt_sp = None
        _agg_body(with_counts, x_hbm, glist_hbm, llist_hbm, len_hbm,
                  zeros_hbm, zcnt_hbm, ones_hbm, out_hbm, cnt_out_hbm,
                  (g0, g1), (l0, l1, l2, l3), (r0, r1), ones_v, lenv, acc_sp, cnt_sp,
                  (i0, i1), (gs0, gs1), (ss0, ss1))

    run = pl.kernel(
        kern,
        out_type=out_type,
        mesh=_MESH,
        scratch_types=scratch,
        compiler_params=_SC_PARAMS,
    )
    out = run(x, glist_d, llist_d, len_d, zeros, zcnt, ones)
    if with_counts:
        return out[0].reshape(N_NODES, EMB), out[1].reshape(N_NODES, CNT_W)
    return out.reshape(N_NODES, EMB)


ROW_B = 1000


def _embed_body(x_ref, sh_ref, sc_ref, W1_ref, b1_ref, W2_ref, b2_ref, o_ref):
    xb = (x_ref[...] + sh_ref[...]) * sc_ref[...]
    h = jnp.dot(xb, W1_ref[...], preferred_element_type=jnp.float32) + b1_ref[...]
    h = jnp.maximum(h, 0.0)
    o = jnp.dot(h, W2_ref[...], preferred_element_type=jnp.float32) + b2_ref[...]
    o_ref[...] = jnp.maximum(o, 0.0)


def _embed(x, sh, sc, W1, b1, W2, b2):
    n, f = x.shape
    full = lambda shape: pl.BlockSpec(shape, lambda i: (0, 0))
    return pl.pallas_call(
        _embed_body,
        grid=(n // ROW_B,),
        in_specs=[
            pl.BlockSpec((ROW_B, f), lambda i: (i, 0)),
            full((1, f)), full((1, f)),
            full((f, EMB)), full((1, EMB)),
            full((EMB, EMB)), full((1, EMB)),
        ],
        out_specs=pl.BlockSpec((ROW_B, EMB), lambda i: (i, 0)),
        out_shape=jax.ShapeDtypeStruct((n, EMB), jnp.float32),
    )(x, sh.reshape(1, f), sc.reshape(1, f), W1, b1.reshape(1, EMB),
      W2, b2.reshape(1, EMB))


def _combine_body(s_ref, c_ref, x_ref, Wl_ref, bl_ref, Wr_ref, o_ref):
    cnt = c_ref[...][:, 0:1]
    mean = s_ref[...] / jnp.maximum(cnt, 1.0)
    o = (jnp.dot(mean, Wl_ref[...], preferred_element_type=jnp.float32)
         + jnp.dot(x_ref[...], Wr_ref[...], preferred_element_type=jnp.float32)
         + bl_ref[...])
    o_ref[...] = jnp.maximum(o, 0.0)


def _combine(sums, cnt, x, Wl, bl, Wr):
    n = sums.shape[0]
    full = lambda shape: pl.BlockSpec(shape, lambda i: (0, 0))
    return pl.pallas_call(
        _combine_body,
        grid=(n // ROW_B,),
        in_specs=[
            pl.BlockSpec((ROW_B, EMB), lambda i: (i, 0)),
            pl.BlockSpec((ROW_B, CNT_W), lambda i: (i, 0)),
            pl.BlockSpec((ROW_B, EMB), lambda i: (i, 0)),
            full((EMB, EMB)), full((1, EMB)), full((EMB, EMB)),
        ],
        out_specs=pl.BlockSpec((ROW_B, EMB), lambda i: (i, 0)),
        out_shape=jax.ShapeDtypeStruct((n, EMB), jnp.float32),
    )(sums, cnt, x, Wl, bl.reshape(1, EMB), Wr)


def kernel(constraint_x, variable_x, edge_attr,
           cons_shift, cons_scale, cons_W1, cons_b1, cons_W2, cons_b2,
           var_shift, var_scale, var_W1, var_b1, var_W2, var_b2,
           edge_shift, edge_scale,
           Wl_c2v_0, bl_c2v_0, Wr_c2v_0, Wl_v2c_0, bl_v2c_0, Wr_v2c_0,
           Wl_c2v_1, bl_c2v_1, Wr_c2v_1, Wl_v2c_1, bl_v2c_1, Wr_v2c_1,
           edge_index):
    src_rs = edge_index[0].reshape(P_CHUNKS, PCH)
    dst_rs = edge_index[1].reshape(P_CHUNKS, PCH)

    gl0, ll0, ln0, gl1, ll1, ln1 = _partition(src_rs, dst_rs)
    c0 = _embed(constraint_x, cons_shift, cons_scale, cons_W1, cons_b1, cons_W2, cons_b2)
    v0 = _embed(variable_x, var_shift, var_scale, var_W1, var_b1, var_W2, var_b2)

    vs0, cnt_v = _aggregate(c0, gl0, ll0, ln0, with_counts=True)
    cs0, cnt_c = _aggregate(v0, gl1, ll1, ln1, with_counts=True)
    v1 = _combine(vs0, cnt_v, v0, Wl_c2v_0, bl_c2v_0, Wr_c2v_0)
    c1 = _combine(cs0, cnt_c, c0, Wl_v2c_0, bl_v2c_0, Wr_v2c_0)

    vs1 = _aggregate(c1, gl0, ll0, ln0, with_counts=False)
    v2 = _combine(vs1, cnt_v, v1, Wl_c2v_1, bl_c2v_1, Wr_c2v_1)
    return v2


# speedup vs baseline: 12.3031x; 1.0319x over previous
import functools

import jax
import jax.numpy as jnp
from jax import lax
from jax.experimental import pallas as pl
from jax.experimental.pallas import tpu as pltpu
from jax.experimental.pallas import tpu_sc as plsc

N_NODES = 50000
N_EDGES = 800000
EMB = 64

CH = 128

PCH = 640
P_CHUNKS = N_EDGES // PCH
CAP = 25728

HALF = N_NODES // 2
ROWS_PER_TILE = 1568
ACC_ROWS = 25008
LAST_TILE_ROWS = ACC_ROWS - 15 * ROWS_PER_TILE
DUMMY = 25000
CNT_W = 8

_MESH = plsc.VectorSubcoreMesh(core_axis_name="c", subcore_axis_name="s")
_SC_PARAMS = pltpu.CompilerParams(use_tc_tiling_on_sc=False)
_SC_PARAMS_NL = pltpu.CompilerParams(use_tc_tiling_on_sc=False,
                                     needs_layout_passes=False)


def _partition_body(src_hbm, dst_hbm, glist0_hbm, llist0_hbm, len0_hbm,
                    glist1_hbm, llist1_hbm, len1_hbm,
                    gbuf, sbuf, g0, l0, g1, l1, lenv):
    sc = lax.axis_index("c")
    t = lax.axis_index("s")
    wid = sc * 16 + t

    for d in (0, 1):
        ga, sa = (src_hbm, dst_hbm) if d == 0 else (dst_hbm, src_hbm)
        glist_hbm = glist0_hbm if d == 0 else glist1_hbm
        llist_hbm = llist0_hbm if d == 0 else llist1_hbm
        len_hbm = len0_hbm if d == 0 else len1_hbm

        def chunk_body(i, ns, ga=ga, sa=sa):
            n0, n1 = ns
            k = wid + 32 * i
            pltpu.sync_copy(ga.at[k], gbuf)
            pltpu.sync_copy(sa.at[k], sbuf)
            lanes = lax.iota(jnp.int32, 16)
            for c in range(PCH // 16):
                g = gbuf[pl.ds(c * 16, 16)]
                s = sbuf[pl.ds(c * 16, 16)]
                m0 = s < HALF
                inc0 = plsc.cumsum(jnp.where(m0, 1, 0).astype(jnp.int32))
                k0 = jnp.sum(jnp.where(m0, 1, 0).astype(jnp.int32))
                pos0 = jnp.where(m0, n0 + inc0 - 1, CAP + lanes)
                pos1 = jnp.where(m0, CAP + lanes, n1 + (lanes - inc0))
                plsc.store_scatter(g0, [pos0], g)
                plsc.store_scatter(l0, [pos0], s)
                plsc.store_scatter(g1, [pos1], g)
                plsc.store_scatter(l1, [pos1], s - HALF)
                n0 = n0 + k0
                n1 = n1 + (16 - k0)
            return (n0, n1)

        n_my = (P_CHUNKS - wid + 31) // 32
        n0, n1 = lax.fori_loop(0, n_my, chunk_body,
                               (jnp.int32(0), jnp.int32(0)))

        dummy_l = jnp.full((16,), DUMMY, jnp.int32)
        dummy_g = jnp.zeros((16,), jnp.int32)
        lanes = lax.iota(jnp.int32, 16)
        for j in range(CH // 16):
            plsc.store_scatter(g0, [n0 + 16 * j + lanes], dummy_g)
            plsc.store_scatter(l0, [n0 + 16 * j + lanes], dummy_l)
            plsc.store_scatter(g1, [n1 + 16 * j + lanes], dummy_g)
            plsc.store_scatter(l1, [n1 + 16 * j + lanes], dummy_l)
        nch0 = (n0 + CH - 1) // CH
        nch1 = (n1 + CH - 1) // CH

        pltpu.sync_copy(g0.at[pl.ds(0, CAP)], glist_hbm.at[wid].at[0])
        pltpu.sync_copy(l0.at[pl.ds(0, CAP)], llist_hbm.at[wid].at[0])
        pltpu.sync_copy(g1.at[pl.ds(0, CAP)], glist_hbm.at[wid].at[1])
        pltpu.sync_copy(l1.at[pl.ds(0, CAP)], llist_hbm.at[wid].at[1])
        lenv[pl.ds(0, 16)] = jnp.zeros((16,), jnp.int32) + nch0
        pltpu.sync_copy(lenv, len_hbm.at[wid].at[0])
        lenv[pl.ds(0, 16)] = jnp.zeros((16,), jnp.int32) + nch1
        pltpu.sync_copy(lenv, len_hbm.at[wid].at[1])


def _partition(src_rs, dst_rs):
    run = pl.kernel(
        _partition_body,
        out_type=(jax.ShapeDtypeStruct((32, 2, CAP), jnp.int32),
                  jax.ShapeDtypeStruct((32, 2, CAP), jnp.int32),
                  jax.ShapeDtypeStruct((32, 2, 16), jnp.int32)) * 2,
        mesh=_MESH,
        scratch_types=[
            pltpu.VMEM((PCH,), jnp.int32),
            pltpu.VMEM((PCH,), jnp.int32),
            pltpu.VMEM((CAP + 16,), jnp.int32),
            pltpu.VMEM((CAP + 16,), jnp.int32),
            pltpu.VMEM((CAP + 16,), jnp.int32),
            pltpu.VMEM((CAP + 16,), jnp.int32),
            pltpu.VMEM((16,), jnp.int32),
        ],
        compiler_params=_SC_PARAMS_NL,
    )
    return run(src_rs, dst_rs)


def _agg_body(with_counts, x_hbm, glist_hbm, llist_hbm, len_hbm, zeros_hbm,
              zcnt_hbm, ones_hbm, out_hbm, cnt_out_hbm, gidx_v, lidx_v,
              rows_v, ones_v, lenv, acc_sp, cnt_sp, isem, gsem, ssem):
    sc = lax.axis_index("c")
    t = lax.axis_index("s")

    def zero_stripe(z_hbm, dst_sp):
        @pl.when(t < 15)
        def _():
            pltpu.sync_copy(z_hbm,
                            dst_sp.at[pl.ds(t * ROWS_PER_TILE, ROWS_PER_TILE)])

        @pl.when(t == 15)
        def _():
            pltpu.sync_copy(z_hbm.at[pl.ds(0, LAST_TILE_ROWS)],
                            dst_sp.at[pl.ds(15 * ROWS_PER_TILE, LAST_TILE_ROWS)])

    zero_stripe(zeros_hbm, acc_sp)
    if with_counts:
        pltpu.sync_copy(ones_hbm, ones_v)
        zero_stripe(zcnt_hbm, cnt_sp)

    pltpu.sync_copy(len_hbm.at[2 * t].at[sc], lenv)
    nch0 = lenv[pl.ds(0, 16)][0]
    pltpu.sync_copy(len_hbm.at[2 * t + 1].at[sc], lenv)
    nch1 = lenv[pl.ds(0, 16)][0]
    ntot = nch0 + nch1
    plsc.subcore_barrier()

    def loc(i):
        p = jnp.where(i < nch0, 2 * t, 2 * t + 1)
        j = jnp.where(i < nch0, i, i - nch0)
        return p, j

    def stage(i, q):
        @pl.when(i < ntot)
        def _():
            p, j = loc(i)
            pltpu.async_copy(glist_hbm.at[p].at[sc].at[pl.ds(j * CH, CH)],
                             gidx_v[q], isem[q % 2])
            pltpu.async_copy(llist_hbm.at[p].at[sc].at[pl.ds(j * CH, CH)],
                             lidx_v[q], isem[q % 2])

    def wait_stage(i, q):
        @pl.when(i < ntot)
        def _():
            p, j = loc(i)
            pltpu.make_async_copy(glist_hbm.at[p].at[sc].at[pl.ds(j * CH, CH)],
                                  gidx_v[q], isem[q % 2]).wait()
            pltpu.make_async_copy(llist_hbm.at[p].at[sc].at[pl.ds(j * CH, CH)],
                                  lidx_v[q], isem[q % 2]).wait()

    def fire_gather(i, b, q):
        @pl.when(i < ntot)
        def _():
            pltpu.async_copy(x_hbm.at[gidx_v[q]], rows_v[b], gsem[b])

    def fire_scatter(i, b, q):
        @pl.when(i < ntot)
        def _():
            pltpu.make_async_copy(x_hbm.at[gidx_v[q]], rows_v[b], gsem[b]).wait()
            pltpu.async_copy(rows_v[b], acc_sp.at[lidx_v[q]], ssem[b],
                             add=True)
            if with_counts:
                pltpu.async_copy(ones_v, cnt_sp.at[lidx_v[q]], ssem[b],
                                 add=True)

    def wait_scatter(i, b, q):
        @pl.when(i < ntot)
        def _():
            pltpu.make_async_copy(rows_v[b], acc_sp.at[lidx_v[q]],
                                  ssem[b]).wait()
            if with_counts:
                pltpu.make_async_copy(ones_v, cnt_sp.at[lidx_v[q]],
                                      ssem[b]).wait()

    stage(0, 0)
    stage(1, 1)

    def body(j, carry):
        for q in range(4):
            i = 4 * j + q
            b = q % 2

            @pl.when(i >= 2)
            def _(i=i, b=b, q=q):
                wait_scatter(i - 2, b, (q + 2) % 4)

            wait_stage(i, q)
            stage(i + 2, (q + 2) % 4)
            fire_gather(i, b, q)

            @pl.when(i >= 1)
            def _(i=i, b=b, q=q):
                fire_scatter(i - 1, 1 - b, (q + 3) % 4)
        return carry

    lax.fori_loop(0, ntot // 4 + 2, body, 0)
    plsc.subcore_barrier()

    def writeback(src_sp, dst_hbm):
        @pl.when(t < 15)
        def _():
            pltpu.sync_copy(src_sp.at[pl.ds(t * ROWS_PER_TILE, ROWS_PER_TILE)],
                            dst_hbm.at[sc].at[pl.ds(t * ROWS_PER_TILE, ROWS_PER_TILE)])

        @pl.when(t == 15)
        def _():
            last = HALF - 15 * ROWS_PER_TILE
            pltpu.sync_copy(src_sp.at[pl.ds(15 * ROWS_PER_TILE, last)],
                            dst_hbm.at[sc].at[pl.ds(15 * ROWS_PER_TILE, last)])

    writeback(acc_sp, out_hbm)
    if with_counts:
        writeback(cnt_sp, cnt_out_hbm)


def _aggregate(x, glist_d, llist_d, len_d, with_counts):
    zeros = jnp.zeros((ROWS_PER_TILE, EMB), jnp.float32)
    zcnt = jnp.zeros((ROWS_PER_TILE, CNT_W), jnp.float32)
    ones = jnp.ones((CH, CNT_W), jnp.float32)
    out_type = jax.ShapeDtypeStruct((2, HALF, EMB), jnp.float32)
    dbuf = lambda shape, dt: [pltpu.VMEM(shape, dt), pltpu.VMEM(shape, dt)]
    sems = lambda: [pltpu.SemaphoreType.DMA, pltpu.SemaphoreType.DMA]
    scratch = (dbuf((CH,), jnp.int32) * 2 +
               dbuf((CH,), jnp.int32) * 2 +
               dbuf((CH, EMB), jnp.float32))
    if with_counts:
        out_type = (out_type, jax.ShapeDtypeStruct((2, HALF, CNT_W), jnp.float32))
        scratch += [pltpu.VMEM((CH, CNT_W), jnp.float32),
                    pltpu.VMEM((16,), jnp.int32),
                    pltpu.VMEM_SHARED((ACC_ROWS, EMB), jnp.float32),
                    pltpu.VMEM_SHARED((ACC_ROWS, CNT_W), jnp.float32)]
    else:
        scratch += [pltpu.VMEM((16,), jnp.int32),
                    pltpu.VMEM_SHARED((ACC_ROWS, EMB), jnp.float32)]
    scratch += sems() + sems() + sems()

    def kern(x_hbm, glist_hbm, llist_hbm, len_hbm, zeros_hbm, zcnt_hbm,
             ones_hbm, *refs):
        if with_counts:
            (out_hbm, cnt_out_hbm, g0, g1, g2, g3, l0, l1, l2, l3, r0, r1,
             ones_v, lenv, acc_sp, cnt_sp, i0, i1, gs0, gs1, ss0, ss1) = refs
        else:
            (out_hbm, g0, g1, g2, g3, l0, l1, l2, l3, r0, r1,
             lenv, acc_sp, i0, i1, gs0, gs1, ss0, ss1) = refs
            cnt_out_hbm = ones_v = cnt_sp = None
        _agg_body(with_counts, x_hbm, glist_hbm, llist_hbm, len_hbm,
                  zeros_hbm, zcnt_hbm, ones_hbm, out_hbm, cnt_out_hbm,
                  (g0, g1, g2, g3), (l0, l1, l2, l3), (r0, r1), ones_v, lenv, acc_sp, cnt_sp,
                  (i0, i1), (gs0, gs1), (ss0, ss1))

    run = pl.kernel(
        kern,
        out_type=out_type,
        mesh=_MESH,
        scratch_types=scratch,
        compiler_params=_SC_PARAMS,
    )
    out = run(x, glist_d, llist_d, len_d, zeros, zcnt, ones)
    if with_counts:
        return out[0].reshape(N_NODES, EMB), out[1].reshape(N_NODES, CNT_W)
    return out.reshape(N_NODES, EMB)


ROW_B = 1000


def _embed_body(x_ref, sh_ref, sc_ref, W1_ref, b1_ref, W2_ref, b2_ref, o_ref):
    xb = (x_ref[...] + sh_ref[...]) * sc_ref[...]
    h = jnp.dot(xb, W1_ref[...], preferred_element_type=jnp.float32) + b1_ref[...]
    h = jnp.maximum(h, 0.0)
    o = jnp.dot(h, W2_ref[...], preferred_element_type=jnp.float32) + b2_ref[...]
    o_ref[...] = jnp.maximum(o, 0.0)


def _embed(x, sh, sc, W1, b1, W2, b2):
    n, f = x.shape
    full = lambda shape: pl.BlockSpec(shape, lambda i: (0, 0))
    return pl.pallas_call(
        _embed_body,
        grid=(n // ROW_B,),
        in_specs=[
            pl.BlockSpec((ROW_B, f), lambda i: (i, 0)),
            full((1, f)), full((1, f)),
            full((f, EMB)), full((1, EMB)),
            full((EMB, EMB)), full((1, EMB)),
        ],
        out_specs=pl.BlockSpec((ROW_B, EMB), lambda i: (i, 0)),
        out_shape=jax.ShapeDtypeStruct((n, EMB), jnp.float32),
    )(x, sh.reshape(1, f), sc.reshape(1, f), W1, b1.reshape(1, EMB),
      W2, b2.reshape(1, EMB))


def _combine_body(s_ref, c_ref, x_ref, Wl_ref, bl_ref, Wr_ref, o_ref):
    cnt = c_ref[...][:, 0:1]
    mean = s_ref[...] / jnp.maximum(cnt, 1.0)
    o = (jnp.dot(mean, Wl_ref[...], preferred_element_type=jnp.float32)
         + jnp.dot(x_ref[...], Wr_ref[...], preferred_element_type=jnp.float32)
         + bl_ref[...])
    o_ref[...] = jnp.maximum(o, 0.0)


def _combine(sums, cnt, x, Wl, bl, Wr):
    n = sums.shape[0]
    full = lambda shape: pl.BlockSpec(shape, lambda i: (0, 0))
    return pl.pallas_call(
        _combine_body,
        grid=(n // ROW_B,),
        in_specs=[
            pl.BlockSpec((ROW_B, EMB), lambda i: (i, 0)),
            pl.BlockSpec((ROW_B, CNT_W), lambda i: (i, 0)),
            pl.BlockSpec((ROW_B, EMB), lambda i: (i, 0)),
            full((EMB, EMB)), full((1, EMB)), full((EMB, EMB)),
        ],
        out_specs=pl.BlockSpec((ROW_B, EMB), lambda i: (i, 0)),
        out_shape=jax.ShapeDtypeStruct((n, EMB), jnp.float32),
    )(sums, cnt, x, Wl, bl.reshape(1, EMB), Wr)


def kernel(constraint_x, variable_x, edge_attr,
           cons_shift, cons_scale, cons_W1, cons_b1, cons_W2, cons_b2,
           var_shift, var_scale, var_W1, var_b1, var_W2, var_b2,
           edge_shift, edge_scale,
           Wl_c2v_0, bl_c2v_0, Wr_c2v_0, Wl_v2c_0, bl_v2c_0, Wr_v2c_0,
           Wl_c2v_1, bl_c2v_1, Wr_c2v_1, Wl_v2c_1, bl_v2c_1, Wr_v2c_1,
           edge_index):
    src_rs = edge_index[0].reshape(P_CHUNKS, PCH)
    dst_rs = edge_index[1].reshape(P_CHUNKS, PCH)

    gl0, ll0, ln0, gl1, ll1, ln1 = _partition(src_rs, dst_rs)
    c0 = _embed(constraint_x, cons_shift, cons_scale, cons_W1, cons_b1, cons_W2, cons_b2)
    v0 = _embed(variable_x, var_shift, var_scale, var_W1, var_b1, var_W2, var_b2)

    vs0, cnt_v = _aggregate(c0, gl0, ll0, ln0, with_counts=True)
    cs0, cnt_c = _aggregate(v0, gl1, ll1, ln1, with_counts=True)
    v1 = _combine(vs0, cnt_v, v0, Wl_c2v_0, bl_c2v_0, Wr_c2v_0)
    c1 = _combine(cs0, cnt_c, c0, Wl_v2c_0, bl_v2c_0, Wr_v2c_0)

    vs1 = _aggregate(c1, gl0, ll0, ln0, with_counts=False)
    v2 = _combine(vs1, cnt_v, v1, Wl_c2v_1, bl_c2v_1, Wr_c2v_1)
    return v2
